# Initial kernel scaffold; baseline (speedup 1.0000x reference)
import functools

import jax
import jax.numpy as jnp
from jax import lax
from jax.experimental import pallas as pl
from jax.experimental.pallas import tpu as pltpu
from jax.experimental.pallas import tpu_sc as plsc

N_NODES = 10000
N_EDGES = 160000
D = 256

NC = 2
NS = 16
NW = NC * NS

CHUNK = 128
E_PAD = 163840
CH_PER_W = E_PAD // (NW * CHUNK)
ROWS_PER_W = E_PAD // NW


def _mm_table_kernel(h_ref, w1_ref, b_ref, o_ref):
    o_ref[...] = lax.dot_general(
        h_ref[...], w1_ref[...],
        (((1,), (1,)), ((), ())),
        preferred_element_type=jnp.float32,
    ) + b_ref[...]


def _mm_edges_kernel(e_ref, g_ref, w2_ref, o_ref):
    o_ref[...] = lax.dot_general(
        e_ref[...], w2_ref[...],
        (((1,), (1,)), ((), ())),
        preferred_element_type=jnp.float32,
    ) + g_ref[...]


@functools.partial(
    pl.kernel,
    out_type=jax.ShapeDtypeStruct((E_PAD, D), jnp.float32),
    mesh=plsc.VectorSubcoreMesh(
        core_axis_name="c", subcore_axis_name="s",
        num_cores=NC, num_subcores=NS,
    ),
    scratch_types=[
        pltpu.VMEM((CH_PER_W, CHUNK), jnp.int32),
        pltpu.VMEM((CHUNK, D), jnp.float32),
        pltpu.SemaphoreType.DMA,
    ],
)
def _sc_gather(table_hbm, idx_hbm, out_hbm, idx_v, rows_v, sem):
    wid = lax.axis_index("s") * NC + lax.axis_index("c")
    base = wid * ROWS_PER_W
    pltpu.sync_copy(idx_hbm.at[wid], idx_v)

    def body(j, carry):
        pltpu.async_copy(table_hbm.at[idx_v.at[j]], rows_v, sem).wait()
        pltpu.sync_copy(rows_v, out_hbm.at[pl.ds(base + j * CHUNK, CHUNK)])
        return carry

    lax.fori_loop(0, CH_PER_W, body, 0)


def kernel(H, E, r_embed, heads, queries, W, b):
    w1 = W[:, :D]
    w2 = W[:, D:]
    b2 = b.reshape(1, D)

    table = pl.pallas_call(
        _mm_table_kernel,
        out_shape=jax.ShapeDtypeStruct((N_NODES, D), jnp.float32),
    )(H, w1, b2)

    heads_pad = jnp.concatenate(
        [heads, jnp.zeros((E_PAD - N_EDGES,), jnp.int32)]
    ).reshape(NW, CH_PER_W, CHUNK)
    gathered = _sc_gather(table, heads_pad)

    blk = 2000
    grid = (N_EDGES // blk,)
    out = pl.pallas_call(
        _mm_edges_kernel,
        grid=grid,
        in_specs=[
            pl.BlockSpec((blk, D), lambda i: (i, 0)),
            pl.BlockSpec((blk, D), lambda i: (i, 0)),
            pl.BlockSpec((D, D), lambda i: (0, 0)),
        ],
        out_specs=pl.BlockSpec((blk, D), lambda i: (i, 0)),
        out_shape=jax.ShapeDtypeStruct((N_EDGES, D), jnp.float32),
    )(E, gathered, w2)
    return out


# speedup vs baseline: 1.0451x; 1.0451x over previous
import functools

import jax
import jax.numpy as jnp
from jax import lax
from jax.experimental import pallas as pl
from jax.experimental.pallas import tpu as pltpu
from jax.experimental.pallas import tpu_sc as plsc

N_NODES = 10000
N_EDGES = 160000
D = 256

NC = 2
NS = 16
NW = NC * NS

CHUNK = 128
E_PAD = 163840
CH_PER_W = E_PAD // (NW * CHUNK)
ROWS_PER_W = E_PAD // NW


def _mm_table_kernel(h_ref, w1_ref, b_ref, o_ref):
    o_ref[...] = lax.dot_general(
        h_ref[...], w1_ref[...],
        (((1,), (1,)), ((), ())),
        preferred_element_type=jnp.float32,
    ) + b_ref[...]


def _mm_edges_kernel(e_ref, g_ref, w2_ref, o_ref):
    o_ref[...] = lax.dot_general(
        e_ref[...], w2_ref[...],
        (((1,), (1,)), ((), ())),
        preferred_element_type=jnp.float32,
    ) + g_ref[...]


@functools.cache
def _make_sc_gather():
    @functools.partial(
        pl.kernel,
        out_type=jax.ShapeDtypeStruct((E_PAD, D), jnp.float32),
        mesh=plsc.VectorSubcoreMesh(
            core_axis_name="c", subcore_axis_name="s",
            num_cores=NC, num_subcores=NS,
        ),
        scratch_types=[
            pltpu.VMEM((CH_PER_W, CHUNK), jnp.int32),
            pltpu.VMEM((CHUNK, D), jnp.float32),
            pltpu.SemaphoreType.DMA,
        ],
    )
    def _sc_gather(table_hbm, idx_hbm, out_hbm, idx_v, rows_v, sem):
        wid = lax.axis_index("s") * NC + lax.axis_index("c")
        base = wid * ROWS_PER_W
        pltpu.sync_copy(idx_hbm.at[wid], idx_v)

        def body(j, carry):
            pltpu.async_copy(table_hbm.at[idx_v.at[j]], rows_v, sem).wait()
            pltpu.sync_copy(rows_v, out_hbm.at[pl.ds(base + j * CHUNK, CHUNK)])
            return carry

        lax.fori_loop(0, CH_PER_W, body, 0)

    return _sc_gather


def kernel(H, E, r_embed, heads, queries, W, b):
    w1 = W[:, :D]
    w2 = W[:, D:]
    b2 = b.reshape(1, D)

    table = pl.pallas_call(
        _mm_table_kernel,
        out_shape=jax.ShapeDtypeStruct((N_NODES, D), jnp.float32),
    )(H, w1, b2)

    heads_pad = jnp.concatenate(
        [heads, jnp.zeros((E_PAD - N_EDGES,), jnp.int32)]
    ).reshape(NW, CH_PER_W, CHUNK)
    gathered = _make_sc_gather()(table, heads_pad)

    blk = 2000
    grid = (N_EDGES // blk,)
    out = pl.pallas_call(
        _mm_edges_kernel,
        grid=grid,
        in_specs=[
            pl.BlockSpec((blk, D), lambda i: (i, 0)),
            pl.BlockSpec((blk, D), lambda i: (i, 0)),
            pl.BlockSpec((D, D), lambda i: (0, 0)),
        ],
        out_specs=pl.BlockSpec((blk, D), lambda i: (i, 0)),
        out_shape=jax.ShapeDtypeStruct((N_EDGES, D), jnp.float32),
    )(E, gathered, w2)
    return out


# speedup vs baseline: 1.1246x; 1.0760x over previous
import functools

import jax
import jax.numpy as jnp
from jax import lax
from jax.experimental import pallas as pl
from jax.experimental.pallas import tpu as pltpu
from jax.experimental.pallas import tpu_sc as plsc

N_NODES = 10000
N_EDGES = 160000
D = 256

NC = 2
NS = 16
NW = NC * NS

CHUNK = 128
E_PAD = 163840
CH_PER_W = E_PAD // (NW * CHUNK)
ROWS_PER_W = E_PAD // NW


def _mm_table_kernel(h_ref, w1_ref, b_ref, o_ref):
    o_ref[...] = lax.dot_general(
        h_ref[...], w1_ref[...],
        (((1,), (1,)), ((), ())),
        preferred_element_type=jnp.float32,
    ) + b_ref[...]


def _mm_edges_kernel(e_ref, g_ref, w2_ref, o_ref):
    o_ref[...] = lax.dot_general(
        e_ref[...], w2_ref[...],
        (((1,), (1,)), ((), ())),
        preferred_element_type=jnp.float32,
    ) + g_ref[...]


@functools.cache
def _make_sc_gather():
    @functools.partial(
        pl.kernel,
        out_type=jax.ShapeDtypeStruct((E_PAD, D), jnp.float32),
        mesh=plsc.VectorSubcoreMesh(
            core_axis_name="c", subcore_axis_name="s",
            num_cores=NC, num_subcores=NS,
        ),
        scratch_types=[
            pltpu.VMEM((CH_PER_W, CHUNK), jnp.int32),
            pltpu.VMEM((CHUNK, D), jnp.float32),
            pltpu.VMEM((CHUNK, D), jnp.float32),
            pltpu.SemaphoreType.DMA,
            pltpu.SemaphoreType.DMA,
        ],
    )
    def _sc_gather(table_hbm, idx_hbm, out_hbm, idx_v, buf_a, buf_b, sem_a,
                   sem_b):
        wid = lax.axis_index("s") * NC + lax.axis_index("c")
        base = wid * ROWS_PER_W
        pltpu.sync_copy(idx_hbm.at[wid], idx_v)

        def gather(j, buf, sem):
            return pltpu.async_copy(table_hbm.at[idx_v.at[j]], buf, sem)

        def store(j, buf):
            pltpu.sync_copy(buf, out_hbm.at[pl.ds(base + j * CHUNK, CHUNK)])

        gather(0, buf_a, sem_a)

        def body(i, carry):
            j = 2 * i
            gather(j + 1, buf_b, sem_b)
            pltpu.make_async_copy(table_hbm.at[idx_v.at[j]], buf_a,
                                  sem_a).wait()
            store(j, buf_a)
            gather(j + 2, buf_a, sem_a)
            pltpu.make_async_copy(table_hbm.at[idx_v.at[j + 1]], buf_b,
                                  sem_b).wait()
            store(j + 1, buf_b)
            return carry

        lax.fori_loop(0, CH_PER_W // 2 - 1, body, 0)
        j = CH_PER_W - 2
        gather(j + 1, buf_b, sem_b)
        pltpu.make_async_copy(table_hbm.at[idx_v.at[j]], buf_a, sem_a).wait()
        store(j, buf_a)
        pltpu.make_async_copy(table_hbm.at[idx_v.at[j + 1]], buf_b,
                              sem_b).wait()
        store(j + 1, buf_b)

    return _sc_gather


def kernel(H, E, r_embed, heads, queries, W, b):
    w1 = W[:, :D]
    w2 = W[:, D:]
    b2 = b.reshape(1, D)

    table = pl.pallas_call(
        _mm_table_kernel,
        out_shape=jax.ShapeDtypeStruct((N_NODES, D), jnp.float32),
    )(H, w1, b2)

    heads_pad = jnp.concatenate(
        [heads, jnp.zeros((E_PAD - N_EDGES,), jnp.int32)]
    ).reshape(NW, CH_PER_W, CHUNK)
    gathered = _make_sc_gather()(table, heads_pad)

    blk = 2000
    grid = (N_EDGES // blk,)
    out = pl.pallas_call(
        _mm_edges_kernel,
        grid=grid,
        in_specs=[
            pl.BlockSpec((blk, D), lambda i: (i, 0)),
            pl.BlockSpec((blk, D), lambda i: (i, 0)),
            pl.BlockSpec((D, D), lambda i: (0, 0)),
        ],
        out_specs=pl.BlockSpec((blk, D), lambda i: (i, 0)),
        out_shape=jax.ShapeDtypeStruct((N_EDGES, D), jnp.float32),
    )(E, gathered, w2)
    return out
